# Initial kernel scaffold; baseline (speedup 1.0000x reference)
import functools

import jax
import jax.numpy as jnp
from jax import lax
from jax.experimental import pallas as pl
from jax.experimental.pallas import tpu as pltpu
from jax.experimental.pallas import tpu_sc as plsc

NC = 2
NS = 16
LANES = 16

S0 = 104


def _pool_body(B, H, D, b_per_w,
               x_hbm, table_hbm, out_hbm,
               idx_v, rows_a, rows_b, out_v, sem_a, sem_b):
  s1 = H - S0
  wid = lax.axis_index("s") * NC + lax.axis_index("c")
  base = wid * b_per_w

  pltpu.sync_copy(x_hbm.at[pl.ds(base * H, b_per_w * H)], idx_v)

  def start(e, rows, sem):
    off = e * H
    pltpu.async_copy(table_hbm.at[idx_v.at[pl.ds(off, S0)]],
                     rows.at[pl.ds(0, S0)], sem)
    pltpu.async_copy(table_hbm.at[idx_v.at[pl.ds(off + S0, s1)]],
                     rows.at[pl.ds(S0, s1)], sem)

  def wait(rows, sem):
    pltpu.make_async_copy(table_hbm.at[idx_v.at[pl.ds(0, S0)]],
                          rows.at[pl.ds(0, S0)], sem).wait()
    pltpu.make_async_copy(table_hbm.at[idx_v.at[pl.ds(0, s1)]],
                          rows.at[pl.ds(S0, s1)], sem).wait()

  def reduce_into(rows, e):
    zero = jnp.zeros((LANES,), jnp.float32)

    def red(i, accs):
      a0, a1, a2, a3 = accs
      for u in range(8):
        j = i * 8 + u
        if u % 2 == 0:
          a0 = a0 + rows[j, 0:16]
          a1 = a1 + rows[j, 16:32]
        else:
          a2 = a2 + rows[j, 0:16]
          a3 = a3 + rows[j, 16:32]
      return a0, a1, a2, a3

    a0, a1, a2, a3 = lax.fori_loop(0, H // 8, red, (zero, zero, zero, zero))
    out_v[e, 0:16] = a0 + a2
    out_v[e, 16:32] = a1 + a3

  start(0, rows_a, sem_a)
  start(1, rows_b, sem_b)

  def body(i, carry):
    e0 = 2 * i
    wait(rows_a, sem_a)
    reduce_into(rows_a, e0)

    @pl.when(e0 + 2 < b_per_w)
    def _():
      start(e0 + 2, rows_a, sem_a)

    wait(rows_b, sem_b)
    reduce_into(rows_b, e0 + 1)

    @pl.when(e0 + 3 < b_per_w)
    def _():
      start(e0 + 3, rows_b, sem_b)

    return carry

  lax.fori_loop(0, b_per_w // 2, body, 0)

  pltpu.sync_copy(out_v, out_hbm.at[pl.ds(base, b_per_w)])


def _head_body(scale, agg_ref, wt_ref, b_ref, out_ref):
  out_ref[...] = (
      jnp.dot(agg_ref[...], wt_ref[...], preferred_element_type=jnp.float32)
      * scale + b_ref[...])


def kernel(x, table, W, b):
  B, H = x.shape
  V, D = table.shape
  OUT = W.shape[0]
  nw = NC * NS
  b_per_w = B // nw

  x_flat = x.astype(jnp.int32).reshape(-1)

  mesh = plsc.VectorSubcoreMesh(core_axis_name="c", subcore_axis_name="s",
                                num_cores=NC, num_subcores=NS)
  sums = pl.kernel(
      functools.partial(_pool_body, B, H, D, b_per_w),
      out_type=jax.ShapeDtypeStruct((B, D), jnp.float32),
      mesh=mesh,
      scratch_types=[
          pltpu.VMEM((b_per_w * H,), jnp.int32),
          pltpu.VMEM((H, D), jnp.float32),
          pltpu.VMEM((H, D), jnp.float32),
          pltpu.VMEM((b_per_w, D), jnp.float32),
          pltpu.SemaphoreType.DMA,
          pltpu.SemaphoreType.DMA,
      ],
  )(x_flat, table)

  out = pl.pallas_call(
      functools.partial(_head_body, 1.0 / H),
      out_shape=jax.ShapeDtypeStruct((B, OUT), jnp.float32),
  )(sums, W.T, b.reshape(1, OUT))
  return out


# speedup vs baseline: 2.3004x; 2.3004x over previous
import functools

import jax
import jax.numpy as jnp
from jax import lax
from jax.experimental import pallas as pl
from jax.experimental.pallas import tpu as pltpu
from jax.experimental.pallas import tpu_sc as plsc

NC = 2
NS = 16
LANES = 16

S0 = 104


def _pool_body(B, H, D, b_per_w,
               x_hbm, table_hbm, out_hbm,
               idx_v, rows_a, rows_b, out_v, sem_a, sem_b):
  s1 = H - S0
  wid = lax.axis_index("s") * NC + lax.axis_index("c")
  base = wid * b_per_w

  pltpu.sync_copy(x_hbm.at[pl.ds(base * H, b_per_w * H)], idx_v)

  def start(e, rows, sem):
    off = e * H
    pltpu.async_copy(table_hbm.at[idx_v.at[pl.ds(off, S0)]],
                     rows.at[pl.ds(0, S0)], sem)
    pltpu.async_copy(table_hbm.at[idx_v.at[pl.ds(off + S0, s1)]],
                     rows.at[pl.ds(S0, s1)], sem)

  def wait(rows, sem):
    pltpu.make_async_copy(table_hbm.at[idx_v.at[pl.ds(0, S0)]],
                          rows.at[pl.ds(0, S0)], sem).wait()
    pltpu.make_async_copy(table_hbm.at[idx_v.at[pl.ds(0, s1)]],
                          rows.at[pl.ds(S0, s1)], sem).wait()

  def reduce_into(rows, e):
    zero = jnp.zeros((LANES,), jnp.float32)

    def red(i, accs):
      a0, a1, a2, a3 = accs
      for u in range(8):
        j = i * 8 + u
        if u % 2 == 0:
          a0 = a0 + rows[j, 0:16]
          a1 = a1 + rows[j, 16:32]
        else:
          a2 = a2 + rows[j, 0:16]
          a3 = a3 + rows[j, 16:32]
      return a0, a1, a2, a3

    a0, a1, a2, a3 = lax.fori_loop(0, H // 8, red, (zero, zero, zero, zero))
    out_v[e, 0:16] = a0 + a2
    out_v[e, 16:32] = a1 + a3

  start(0, rows_a, sem_a)
  start(1, rows_b, sem_b)

  def body(i, carry):
    e0 = 2 * i
    wait(rows_a, sem_a)
    reduce_into(rows_a, e0)

    @pl.when(e0 + 2 < b_per_w)
    def _():
      start(e0 + 2, rows_a, sem_a)

    wait(rows_b, sem_b)
    reduce_into(rows_b, e0 + 1)

    @pl.when(e0 + 3 < b_per_w)
    def _():
      start(e0 + 3, rows_b, sem_b)

    return carry

  lax.fori_loop(0, b_per_w // 2, body, 0)

  pltpu.sync_copy(out_v, out_hbm.at[pl.ds(base, b_per_w)])


def _head_body(scale, agg_ref, wt_ref, b_ref, out_ref):
  out_ref[...] = (
      jnp.dot(agg_ref[...], wt_ref[...], preferred_element_type=jnp.float32)
      * scale + b_ref[...])


def kernel(x, table, W, b):
  B, H = x.shape
  V, D = table.shape
  OUT = W.shape[0]
  nw = NC * NS
  b_per_w = B // nw

  x_flat = x.astype(jnp.int32).reshape(-1)

  mesh = plsc.VectorSubcoreMesh(core_axis_name="c", subcore_axis_name="s",
                                num_cores=NC, num_subcores=NS)
  sums = pl.kernel(
      functools.partial(_pool_body, B, H, D, b_per_w),
      out_type=jax.ShapeDtypeStruct((B, D), jnp.float32),
      mesh=mesh,
      compiler_params=pltpu.CompilerParams(use_tc_tiling_on_sc=False),
      scratch_types=[
          pltpu.VMEM((b_per_w * H,), jnp.int32),
          pltpu.VMEM((H, D), jnp.float32),
          pltpu.VMEM((H, D), jnp.float32),
          pltpu.VMEM((b_per_w, D), jnp.float32),
          pltpu.SemaphoreType.DMA,
          pltpu.SemaphoreType.DMA,
      ],
  )(x_flat, table)

  out = pl.pallas_call(
      functools.partial(_head_body, 1.0 / H),
      out_shape=jax.ShapeDtypeStruct((B, OUT), jnp.float32),
  )(sums, W.T, b.reshape(1, OUT))
  return out


# speedup vs baseline: 2.6096x; 1.1344x over previous
import functools

import jax
import jax.numpy as jnp
from jax import lax
from jax.experimental import pallas as pl
from jax.experimental.pallas import tpu as pltpu
from jax.experimental.pallas import tpu_sc as plsc

NC = 2
NS = 16
LANES = 16

S0 = 104


def _pool_body(B, H, D, b_per_w,
               x_hbm, table_hbm, out_hbm,
               idx_v, rows_a, rows_b, out_v, sem_a, sem_b):
  s1 = H - S0
  wid = lax.axis_index("s") * NC + lax.axis_index("c")
  base = wid * b_per_w

  pltpu.sync_copy(x_hbm.at[pl.ds(base * H, b_per_w * H)], idx_v)

  def tbody(i, carry):
    for u in range(4):
      o = (i * 4 + u) * LANES
      v = idx_v[pl.ds(o, LANES)]
      w = v & 2047
      idx_v[pl.ds(o, LANES)] = (v - w) + ((w & 511) << 2) + (w >> 9)
    return carry

  lax.fori_loop(0, (b_per_w * H) // (4 * LANES), tbody, 0)

  def start(e, rows, sem):
    off = e * H
    pltpu.async_copy(table_hbm.at[idx_v.at[pl.ds(off, S0)]],
                     rows.at[pl.ds(0, S0)], sem)
    pltpu.async_copy(table_hbm.at[idx_v.at[pl.ds(off + S0, s1)]],
                     rows.at[pl.ds(S0, s1)], sem)

  def wait(rows, sem):
    pltpu.make_async_copy(table_hbm.at[idx_v.at[pl.ds(0, S0)]],
                          rows.at[pl.ds(0, S0)], sem).wait()
    pltpu.make_async_copy(table_hbm.at[idx_v.at[pl.ds(0, s1)]],
                          rows.at[pl.ds(S0, s1)], sem).wait()

  def reduce_into(rows, e):
    zero = jnp.zeros((LANES,), jnp.float32)

    def red(i, accs):
      a0, a1, a2, a3 = accs
      for u in range(8):
        j = i * 8 + u
        if u % 2 == 0:
          a0 = a0 + rows[j, 0:16]
          a1 = a1 + rows[j, 16:32]
        else:
          a2 = a2 + rows[j, 0:16]
          a3 = a3 + rows[j, 16:32]
      return a0, a1, a2, a3

    a0, a1, a2, a3 = lax.fori_loop(0, H // 8, red, (zero, zero, zero, zero))
    out_v[e, 0:16] = a0 + a2
    out_v[e, 16:32] = a1 + a3

  start(0, rows_a, sem_a)
  start(1, rows_b, sem_b)

  def body(i, carry):
    e0 = 2 * i
    wait(rows_a, sem_a)
    reduce_into(rows_a, e0)

    @pl.when(e0 + 2 < b_per_w)
    def _():
      start(e0 + 2, rows_a, sem_a)

    wait(rows_b, sem_b)
    reduce_into(rows_b, e0 + 1)

    @pl.when(e0 + 3 < b_per_w)
    def _():
      start(e0 + 3, rows_b, sem_b)

    return carry

  lax.fori_loop(0, b_per_w // 2, body, 0)

  pltpu.sync_copy(out_v, out_hbm.at[pl.ds(base, b_per_w)])


def _pack_body(tt_ref, out_ref):
  t = tt_ref[...].T
  for k in range(4):
    out_ref[:, 32 * k:32 * (k + 1)] = t[512 * k:512 * (k + 1), :]


def _head_body(scale, agg_ref, wt_ref, b_ref, out_ref):
  out_ref[...] = (
      jnp.dot(agg_ref[...], wt_ref[...], preferred_element_type=jnp.float32)
      * scale + b_ref[...])


def kernel(x, table, W, b):
  B, H = x.shape
  V, D = table.shape
  OUT = W.shape[0]
  nw = NC * NS
  b_per_w = B // nw

  x_flat = x.astype(jnp.int32).reshape(-1)

  C = 2048
  grid = (V + C - 1) // C
  packed = pl.pallas_call(
      _pack_body,
      grid=(grid,),
      in_specs=[pl.BlockSpec((D, C), lambda j: (0, j))],
      out_specs=pl.BlockSpec((C // 4, 128), lambda j: (j, 0)),
      out_shape=jax.ShapeDtypeStruct((grid * (C // 4), 128), jnp.float32),
  )(table.T)
  v_pad = grid * C
  table_lin = packed.reshape(v_pad, D)

  mesh = plsc.VectorSubcoreMesh(core_axis_name="c", subcore_axis_name="s",
                                num_cores=NC, num_subcores=NS)
  sums = pl.kernel(
      functools.partial(_pool_body, B, H, D, b_per_w),
      out_type=jax.ShapeDtypeStruct((B, D), jnp.float32),
      mesh=mesh,
      compiler_params=pltpu.CompilerParams(use_tc_tiling_on_sc=False),
      scratch_types=[
          pltpu.VMEM((b_per_w * H,), jnp.int32),
          pltpu.VMEM((H, D), jnp.float32),
          pltpu.VMEM((H, D), jnp.float32),
          pltpu.VMEM((b_per_w, D), jnp.float32),
          pltpu.SemaphoreType.DMA,
          pltpu.SemaphoreType.DMA,
      ],
  )(x_flat, table_lin)

  out = pl.pallas_call(
      functools.partial(_head_body, 1.0 / H),
      out_shape=jax.ShapeDtypeStruct((B, OUT), jnp.float32),
  )(sums, W.T, b.reshape(1, OUT))
  return out


# speedup vs baseline: 3.1590x; 1.2105x over previous
import functools

import jax
import jax.numpy as jnp
from jax import lax
from jax.experimental import pallas as pl
from jax.experimental.pallas import tpu as pltpu
from jax.experimental.pallas import tpu_sc as plsc

NC = 2
NS = 16
LANES = 16

S0 = 104
C = 2048


def _pack_body(tt_ref, out_ref):
  stacked = jnp.concatenate(
      [tt_ref[:, 512 * k:512 * (k + 1)] for k in range(4)], axis=0)
  out_ref[...] = stacked.T


def _pool_body(B, H, b_per_w,
               x_hbm, table_hbm, out_hbm,
               idx_v, rows_a, rows_b, out_v, sem_a, sem_b):
  s1 = H - S0
  wid = lax.axis_index("s") * NC + lax.axis_index("c")
  base = wid * b_per_w

  pltpu.sync_copy(x_hbm.at[pl.ds(base * H, b_per_w * H)], idx_v)

  def tbody(i, carry):
    for u in range(4):
      o = (i * 4 + u) * LANES
      v = idx_v[pl.ds(o, LANES)]
      w = v & (C - 1)
      idx_v[pl.ds(o, LANES)] = (v - w) + ((w & 511) << 2) + (w >> 9)
    return carry

  lax.fori_loop(0, (b_per_w * H) // (4 * LANES), tbody, 0)

  def start(e, rows, sem):
    off = e * H
    pltpu.async_copy(table_hbm.at[idx_v.at[pl.ds(off, S0)]],
                     rows.at[pl.ds(0, S0)], sem)
    pltpu.async_copy(table_hbm.at[idx_v.at[pl.ds(off + S0, s1)]],
                     rows.at[pl.ds(S0, s1)], sem)

  def wait(rows, sem):
    pltpu.make_async_copy(table_hbm.at[idx_v.at[pl.ds(0, S0)]],
                          rows.at[pl.ds(0, S0)], sem).wait()
    pltpu.make_async_copy(table_hbm.at[idx_v.at[pl.ds(0, s1)]],
                          rows.at[pl.ds(S0, s1)], sem).wait()

  def reduce_into(rows, e):
    zero = jnp.zeros((LANES,), jnp.float32)

    def red(i, accs):
      a0, a1, a2, a3 = accs
      for u in range(8):
        j = i * 8 + u
        if u % 2 == 0:
          a0 = a0 + rows[j, 0:16]
          a1 = a1 + rows[j, 16:32]
        else:
          a2 = a2 + rows[j, 0:16]
          a3 = a3 + rows[j, 16:32]
      return a0, a1, a2, a3

    a0, a1, a2, a3 = lax.fori_loop(0, H // 8, red, (zero, zero, zero, zero))
    out_v[e, 0:16] = a0 + a2
    out_v[e, 16:32] = a1 + a3

  start(0, rows_a, sem_a)
  start(1, rows_b, sem_b)

  def body(i, carry):
    e0 = 2 * i
    wait(rows_a, sem_a)
    reduce_into(rows_a, e0)

    @pl.when(e0 + 2 < b_per_w)
    def _():
      start(e0 + 2, rows_a, sem_a)

    wait(rows_b, sem_b)
    reduce_into(rows_b, e0 + 1)

    @pl.when(e0 + 3 < b_per_w)
    def _():
      start(e0 + 3, rows_b, sem_b)

    return carry

  lax.fori_loop(0, b_per_w // 2, body, 0)

  pltpu.sync_copy(out_v, out_hbm.at[pl.ds(base, b_per_w)])


def _head_body(scale, agg_ref, wt_ref, b_ref, out_ref):
  out_ref[...] = (
      jnp.dot(agg_ref[...], wt_ref[...], preferred_element_type=jnp.float32)
      * scale + b_ref[...])


def kernel(x, table, W, b):
  B, H = x.shape
  V, D = table.shape
  OUT = W.shape[0]
  nw = NC * NS
  b_per_w = B // nw

  x_flat = x.astype(jnp.int32).reshape(-1)

  grid = (V + C - 1) // C
  packed = pl.pallas_call(
      _pack_body,
      grid=(grid,),
      in_specs=[pl.BlockSpec((D, C), lambda j: (0, j))],
      out_specs=pl.BlockSpec((C // 4, 128), lambda j: (j, 0)),
      out_shape=jax.ShapeDtypeStruct((grid * (C // 4), 128), jnp.float32),
  )(table.T)
  v_pad = grid * C
  table_lin = packed.reshape(v_pad, D)

  mesh = plsc.VectorSubcoreMesh(core_axis_name="c", subcore_axis_name="s",
                                num_cores=NC, num_subcores=NS)
  sums = pl.kernel(
      functools.partial(_pool_body, B, H, b_per_w),
      out_type=jax.ShapeDtypeStruct((B, D), jnp.float32),
      mesh=mesh,
      compiler_params=pltpu.CompilerParams(use_tc_tiling_on_sc=False),
      scratch_types=[
          pltpu.VMEM((b_per_w * H,), jnp.int32),
          pltpu.VMEM((H, D), jnp.float32),
          pltpu.VMEM((H, D), jnp.float32),
          pltpu.VMEM((b_per_w, D), jnp.float32),
          pltpu.SemaphoreType.DMA,
          pltpu.SemaphoreType.DMA,
      ],
  )(x_flat, table_lin)

  out = pl.pallas_call(
      functools.partial(_head_body, 1.0 / H),
      out_shape=jax.ShapeDtypeStruct((B, OUT), jnp.float32),
  )(sums, W.T, b.reshape(1, OUT))
  return out


# speedup vs baseline: 5.4363x; 1.7209x over previous
import functools

import jax
import jax.numpy as jnp
from jax import lax
from jax.experimental import pallas as pl
from jax.experimental.pallas import tpu as pltpu
from jax.experimental.pallas import tpu_sc as plsc

NC = 2
NS = 16
LANES = 16

S0 = 104
C = 8192


def _pack_body(tt_ref, out_ref):
  n = C // 4
  stacked = jnp.concatenate(
      [tt_ref[:, n * k:n * (k + 1)] for k in range(4)], axis=0)
  out_ref[...] = stacked.T


def _pool_body(B, H, b_per_w,
               x_hbm, table_hbm, out_hbm,
               idx_v, rows_a, rows_b, out_v, sem_a, sem_b):
  s1 = H - S0
  wid = lax.axis_index("s") * NC + lax.axis_index("c")
  base = wid * b_per_w

  pltpu.sync_copy(x_hbm.at[pl.ds(base * H, b_per_w * H)], idx_v)

  n = C // 4
  sh = n.bit_length() - 1

  def tbody(i, carry):
    for u in range(4):
      o = (i * 4 + u) * LANES
      v = idx_v[pl.ds(o, LANES)]
      w = v & (C - 1)
      idx_v[pl.ds(o, LANES)] = (v - w) + ((w & (n - 1)) << 2) + (w >> sh)
    return carry

  lax.fori_loop(0, (b_per_w * H) // (4 * LANES), tbody, 0)

  def start(e, rows, sem):
    off = e * H
    pltpu.async_copy(table_hbm.at[idx_v.at[pl.ds(off, S0)]],
                     rows.at[pl.ds(0, S0)], sem)
    pltpu.async_copy(table_hbm.at[idx_v.at[pl.ds(off + S0, s1)]],
                     rows.at[pl.ds(S0, s1)], sem)

  def wait(rows, sem):
    pltpu.make_async_copy(table_hbm.at[idx_v.at[pl.ds(0, S0)]],
                          rows.at[pl.ds(0, S0)], sem).wait()
    pltpu.make_async_copy(table_hbm.at[idx_v.at[pl.ds(0, s1)]],
                          rows.at[pl.ds(S0, s1)], sem).wait()

  def reduce_into(rows, e):
    zero = jnp.zeros((LANES,), jnp.float32)

    def red(i, accs):
      a0, a1, a2, a3 = accs
      for u in range(8):
        j = i * 8 + u
        if u % 2 == 0:
          a0 = a0 + rows[j, 0:16]
          a1 = a1 + rows[j, 16:32]
        else:
          a2 = a2 + rows[j, 0:16]
          a3 = a3 + rows[j, 16:32]
      return a0, a1, a2, a3

    a0, a1, a2, a3 = lax.fori_loop(0, H // 8, red, (zero, zero, zero, zero))
    out_v[e, 0:16] = a0 + a2
    out_v[e, 16:32] = a1 + a3

  start(0, rows_a, sem_a)
  start(1, rows_b, sem_b)

  def body(i, carry):
    e0 = 2 * i
    wait(rows_a, sem_a)
    reduce_into(rows_a, e0)

    @pl.when(e0 + 2 < b_per_w)
    def _():
      start(e0 + 2, rows_a, sem_a)

    wait(rows_b, sem_b)
    reduce_into(rows_b, e0 + 1)

    @pl.when(e0 + 3 < b_per_w)
    def _():
      start(e0 + 3, rows_b, sem_b)

    return carry

  lax.fori_loop(0, b_per_w // 2, body, 0)

  pltpu.sync_copy(out_v, out_hbm.at[pl.ds(base, b_per_w)])


def _head_body(scale, agg_ref, wt_ref, b_ref, out_ref):
  out_ref[...] = (
      jnp.dot(agg_ref[...], wt_ref[...], preferred_element_type=jnp.float32)
      * scale + b_ref[...])


def kernel(x, table, W, b):
  B, H = x.shape
  V, D = table.shape
  OUT = W.shape[0]
  nw = NC * NS
  b_per_w = B // nw

  x_flat = x.astype(jnp.int32).reshape(-1)

  grid = (V + C - 1) // C
  packed = pl.pallas_call(
      _pack_body,
      grid=(grid,),
      in_specs=[pl.BlockSpec((D, C), lambda j: (0, j))],
      out_specs=pl.BlockSpec((C // 4, 128), lambda j: (j, 0)),
      out_shape=jax.ShapeDtypeStruct((grid * (C // 4), 128), jnp.float32),
  )(table.T)
  v_pad = grid * C
  table_lin = packed.reshape(v_pad, D)

  mesh = plsc.VectorSubcoreMesh(core_axis_name="c", subcore_axis_name="s",
                                num_cores=NC, num_subcores=NS)
  sums = pl.kernel(
      functools.partial(_pool_body, B, H, b_per_w),
      out_type=jax.ShapeDtypeStruct((B, D), jnp.float32),
      mesh=mesh,
      compiler_params=pltpu.CompilerParams(use_tc_tiling_on_sc=False),
      scratch_types=[
          pltpu.VMEM((b_per_w * H,), jnp.int32),
          pltpu.VMEM((H, D), jnp.float32),
          pltpu.VMEM((H, D), jnp.float32),
          pltpu.VMEM((b_per_w, D), jnp.float32),
          pltpu.SemaphoreType.DMA,
          pltpu.SemaphoreType.DMA,
      ],
  )(x_flat, table_lin)

  out = pl.pallas_call(
      functools.partial(_head_body, 1.0 / H),
      out_shape=jax.ShapeDtypeStruct((B, OUT), jnp.float32),
  )(sums, W.T, b.reshape(1, OUT))
  return out


# speedup vs baseline: 6.8915x; 1.2677x over previous
import functools

import jax
import jax.numpy as jnp
from jax import lax
from jax.experimental import pallas as pl
from jax.experimental.pallas import tpu as pltpu
from jax.experimental.pallas import tpu_sc as plsc

NC = 2
NS = 16
LANES = 16

S0 = 104
C = 32768


def _pack_body(tt_ref, out_ref):
  n = C // 4
  stacked = jnp.concatenate(
      [tt_ref[:, n * k:n * (k + 1)] for k in range(4)], axis=0)
  out_ref[...] = stacked.T


def _pool_body(B, H, b_per_w,
               x_hbm, table_hbm, out_hbm,
               idx_v, rows_a, rows_b, out_v, sem_a, sem_b):
  s1 = H - S0
  wid = lax.axis_index("s") * NC + lax.axis_index("c")
  base = wid * b_per_w

  pltpu.sync_copy(x_hbm.at[pl.ds(base * H, b_per_w * H)], idx_v)

  n = C // 4
  sh = n.bit_length() - 1

  def tbody(i, carry):
    for u in range(4):
      o = (i * 4 + u) * LANES
      v = idx_v[pl.ds(o, LANES)]
      w = v & (C - 1)
      idx_v[pl.ds(o, LANES)] = (v - w) + ((w & (n - 1)) << 2) + (w >> sh)
    return carry

  lax.fori_loop(0, (b_per_w * H) // (4 * LANES), tbody, 0)

  def start(e, rows, sem):
    off = e * H
    pltpu.async_copy(table_hbm.at[idx_v.at[pl.ds(off, S0)]],
                     rows.at[pl.ds(0, S0)], sem)
    pltpu.async_copy(table_hbm.at[idx_v.at[pl.ds(off + S0, s1)]],
                     rows.at[pl.ds(S0, s1)], sem)

  def wait(rows, sem):
    pltpu.make_async_copy(table_hbm.at[idx_v.at[pl.ds(0, S0)]],
                          rows.at[pl.ds(0, S0)], sem).wait()
    pltpu.make_async_copy(table_hbm.at[idx_v.at[pl.ds(0, s1)]],
                          rows.at[pl.ds(S0, s1)], sem).wait()

  def reduce_into(rows, e):
    zero = jnp.zeros((LANES,), jnp.float32)

    def red(i, accs):
      a0, a1, a2, a3 = accs
      for u in range(8):
        j = i * 8 + u
        if u % 2 == 0:
          a0 = a0 + rows[j, 0:16]
          a1 = a1 + rows[j, 16:32]
        else:
          a2 = a2 + rows[j, 0:16]
          a3 = a3 + rows[j, 16:32]
      return a0, a1, a2, a3

    a0, a1, a2, a3 = lax.fori_loop(0, H // 8, red, (zero, zero, zero, zero))
    out_v[e, 0:16] = a0 + a2
    out_v[e, 16:32] = a1 + a3

  start(0, rows_a, sem_a)
  start(1, rows_b, sem_b)

  def body(i, carry):
    e0 = 2 * i
    wait(rows_a, sem_a)
    reduce_into(rows_a, e0)

    @pl.when(e0 + 2 < b_per_w)
    def _():
      start(e0 + 2, rows_a, sem_a)

    wait(rows_b, sem_b)
    reduce_into(rows_b, e0 + 1)

    @pl.when(e0 + 3 < b_per_w)
    def _():
      start(e0 + 3, rows_b, sem_b)

    return carry

  lax.fori_loop(0, b_per_w // 2, body, 0)

  pltpu.sync_copy(out_v, out_hbm.at[pl.ds(base, b_per_w)])


def _head_body(scale, agg_ref, wt_ref, b_ref, out_ref):
  out_ref[...] = (
      jnp.dot(agg_ref[...], wt_ref[...], preferred_element_type=jnp.float32)
      * scale + b_ref[...])


def kernel(x, table, W, b):
  B, H = x.shape
  V, D = table.shape
  OUT = W.shape[0]
  nw = NC * NS
  b_per_w = B // nw

  x_flat = x.astype(jnp.int32).reshape(-1)

  grid = (V + C - 1) // C
  packed = pl.pallas_call(
      _pack_body,
      grid=(grid,),
      in_specs=[pl.BlockSpec((D, C), lambda j: (0, j))],
      out_specs=pl.BlockSpec((C // 4, 128), lambda j: (j, 0)),
      out_shape=jax.ShapeDtypeStruct((grid * (C // 4), 128), jnp.float32),
  )(table.T)
  v_pad = grid * C
  table_lin = packed.reshape(v_pad, D)

  mesh = plsc.VectorSubcoreMesh(core_axis_name="c", subcore_axis_name="s",
                                num_cores=NC, num_subcores=NS)
  sums = pl.kernel(
      functools.partial(_pool_body, B, H, b_per_w),
      out_type=jax.ShapeDtypeStruct((B, D), jnp.float32),
      mesh=mesh,
      compiler_params=pltpu.CompilerParams(use_tc_tiling_on_sc=False),
      scratch_types=[
          pltpu.VMEM((b_per_w * H,), jnp.int32),
          pltpu.VMEM((H, D), jnp.float32),
          pltpu.VMEM((H, D), jnp.float32),
          pltpu.VMEM((b_per_w, D), jnp.float32),
          pltpu.SemaphoreType.DMA,
          pltpu.SemaphoreType.DMA,
      ],
  )(x_flat, table_lin)

  out = pl.pallas_call(
      functools.partial(_head_body, 1.0 / H),
      out_shape=jax.ShapeDtypeStruct((B, OUT), jnp.float32),
  )(sums, W.T, b.reshape(1, OUT))
  return out


# speedup vs baseline: 6.9452x; 1.0078x over previous
import functools

import jax
import jax.numpy as jnp
from jax import lax
from jax.experimental import pallas as pl
from jax.experimental.pallas import tpu as pltpu
from jax.experimental.pallas import tpu_sc as plsc

NC = 2
NS = 16
LANES = 16

S0 = 104
C = 65536


def _pack_body(tt_ref, out_ref):
  n = C // 4
  stacked = jnp.concatenate(
      [tt_ref[:, n * k:n * (k + 1)] for k in range(4)], axis=0)
  out_ref[...] = stacked.T


def _pool_body(B, H, b_per_w,
               x_hbm, table_hbm, out_hbm,
               idx_v, rows_a, rows_b, out_v, sem_a, sem_b):
  s1 = H - S0
  wid = lax.axis_index("s") * NC + lax.axis_index("c")
  base = wid * b_per_w

  pltpu.sync_copy(x_hbm.at[pl.ds(base * H, b_per_w * H)], idx_v)

  n = C // 4
  sh = n.bit_length() - 1

  def tbody(i, carry):
    for u in range(4):
      o = (i * 4 + u) * LANES
      v = idx_v[pl.ds(o, LANES)]
      w = v & (C - 1)
      idx_v[pl.ds(o, LANES)] = (v - w) + ((w & (n - 1)) << 2) + (w >> sh)
    return carry

  lax.fori_loop(0, (b_per_w * H) // (4 * LANES), tbody, 0)

  def start(e, rows, sem):
    off = e * H
    pltpu.async_copy(table_hbm.at[idx_v.at[pl.ds(off, S0)]],
                     rows.at[pl.ds(0, S0)], sem)
    pltpu.async_copy(table_hbm.at[idx_v.at[pl.ds(off + S0, s1)]],
                     rows.at[pl.ds(S0, s1)], sem)

  def wait(rows, sem):
    pltpu.make_async_copy(table_hbm.at[idx_v.at[pl.ds(0, S0)]],
                          rows.at[pl.ds(0, S0)], sem).wait()
    pltpu.make_async_copy(table_hbm.at[idx_v.at[pl.ds(0, s1)]],
                          rows.at[pl.ds(S0, s1)], sem).wait()

  def reduce_into(rows, e):
    zero = jnp.zeros((LANES,), jnp.float32)

    def red(i, accs):
      a0, a1, a2, a3 = accs
      for u in range(8):
        j = i * 8 + u
        if u % 2 == 0:
          a0 = a0 + rows[j, 0:16]
          a1 = a1 + rows[j, 16:32]
        else:
          a2 = a2 + rows[j, 0:16]
          a3 = a3 + rows[j, 16:32]
      return a0, a1, a2, a3

    a0, a1, a2, a3 = lax.fori_loop(0, H // 8, red, (zero, zero, zero, zero))
    out_v[e, 0:16] = a0 + a2
    out_v[e, 16:32] = a1 + a3

  start(0, rows_a, sem_a)
  start(1, rows_b, sem_b)

  def body(i, carry):
    e0 = 2 * i
    wait(rows_a, sem_a)
    reduce_into(rows_a, e0)

    @pl.when(e0 + 2 < b_per_w)
    def _():
      start(e0 + 2, rows_a, sem_a)

    wait(rows_b, sem_b)
    reduce_into(rows_b, e0 + 1)

    @pl.when(e0 + 3 < b_per_w)
    def _():
      start(e0 + 3, rows_b, sem_b)

    return carry

  lax.fori_loop(0, b_per_w // 2, body, 0)

  pltpu.sync_copy(out_v, out_hbm.at[pl.ds(base, b_per_w)])


def _head_body(scale, agg_ref, wt_ref, b_ref, out_ref):
  out_ref[...] = (
      jnp.dot(agg_ref[...], wt_ref[...], preferred_element_type=jnp.float32)
      * scale + b_ref[...])


def kernel(x, table, W, b):
  B, H = x.shape
  V, D = table.shape
  OUT = W.shape[0]
  nw = NC * NS
  b_per_w = B // nw

  x_flat = x.astype(jnp.int32).reshape(-1)

  grid = (V + C - 1) // C
  packed = pl.pallas_call(
      _pack_body,
      grid=(grid,),
      in_specs=[pl.BlockSpec((D, C), lambda j: (0, j))],
      out_specs=pl.BlockSpec((C // 4, 128), lambda j: (j, 0)),
      out_shape=jax.ShapeDtypeStruct((grid * (C // 4), 128), jnp.float32),
  )(table.T)
  v_pad = grid * C
  table_lin = packed.reshape(v_pad, D)

  mesh = plsc.VectorSubcoreMesh(core_axis_name="c", subcore_axis_name="s",
                                num_cores=NC, num_subcores=NS)
  sums = pl.kernel(
      functools.partial(_pool_body, B, H, b_per_w),
      out_type=jax.ShapeDtypeStruct((B, D), jnp.float32),
      mesh=mesh,
      compiler_params=pltpu.CompilerParams(use_tc_tiling_on_sc=False),
      scratch_types=[
          pltpu.VMEM((b_per_w * H,), jnp.int32),
          pltpu.VMEM((H, D), jnp.float32),
          pltpu.VMEM((H, D), jnp.float32),
          pltpu.VMEM((b_per_w, D), jnp.float32),
          pltpu.SemaphoreType.DMA,
          pltpu.SemaphoreType.DMA,
      ],
  )(x_flat, table_lin)

  out = pl.pallas_call(
      functools.partial(_head_body, 1.0 / H),
      out_shape=jax.ShapeDtypeStruct((B, OUT), jnp.float32),
  )(sums, W.T, b.reshape(1, OUT))
  return out


# speedup vs baseline: 8.3485x; 1.2021x over previous
import functools

import jax
import jax.numpy as jnp
from jax import lax
from jax.experimental import pallas as pl
from jax.experimental.pallas import tpu as pltpu
from jax.experimental.pallas import tpu_sc as plsc

NC = 2
NS = 16
LANES = 16

S0 = 104
C = 65536


def _pack_body(tt_ref, out_ref):
  n = C // 4
  stacked = jnp.concatenate(
      [tt_ref[:, n * k:n * (k + 1)] for k in range(4)], axis=0)
  out_ref[...] = stacked.T


def _pool_body(B, H, b_per_w,
               x_hbm, table_hbm, out_hbm,
               idx_v, rows_a, rows_b, rows_c, rows_d, out_v,
               sem_a, sem_b, sem_c, sem_d):
  s1 = H - S0
  wid = lax.axis_index("s") * NC + lax.axis_index("c")
  base = wid * b_per_w

  pltpu.sync_copy(x_hbm.at[pl.ds(base * H, b_per_w * H)], idx_v)

  n = C // 4
  sh = n.bit_length() - 1

  def tbody(i, carry):
    for u in range(4):
      o = (i * 4 + u) * LANES
      v = idx_v[pl.ds(o, LANES)]
      w = v & (C - 1)
      idx_v[pl.ds(o, LANES)] = (v - w) + ((w & (n - 1)) << 2) + (w >> sh)
    return carry

  lax.fori_loop(0, (b_per_w * H) // (4 * LANES), tbody, 0)

  def start(e, rows, sem):
    off = e * H
    pltpu.async_copy(table_hbm.at[idx_v.at[pl.ds(off, S0)]],
                     rows.at[pl.ds(0, S0)], sem)
    pltpu.async_copy(table_hbm.at[idx_v.at[pl.ds(off + S0, s1)]],
                     rows.at[pl.ds(S0, s1)], sem)

  def wait(rows, sem):
    pltpu.make_async_copy(table_hbm.at[idx_v.at[pl.ds(0, S0)]],
                          rows.at[pl.ds(0, S0)], sem).wait()
    pltpu.make_async_copy(table_hbm.at[idx_v.at[pl.ds(0, s1)]],
                          rows.at[pl.ds(S0, s1)], sem).wait()

  def reduce_into(rows, e):
    zero = jnp.zeros((LANES,), jnp.float32)

    def red(i, accs):
      a0, a1, a2, a3 = accs
      for u in range(8):
        j = i * 8 + u
        if u % 2 == 0:
          a0 = a0 + rows[j, 0:16]
          a1 = a1 + rows[j, 16:32]
        else:
          a2 = a2 + rows[j, 0:16]
          a3 = a3 + rows[j, 16:32]
      return a0, a1, a2, a3

    a0, a1, a2, a3 = lax.fori_loop(0, H // 8, red, (zero, zero, zero, zero))
    out_v[e, 0:16] = a0 + a2
    out_v[e, 16:32] = a1 + a3

  bufs = (rows_a, rows_b, rows_c, rows_d)
  sems = (sem_a, sem_b, sem_c, sem_d)
  nbuf = len(bufs)
  for u in range(nbuf):
    start(u, bufs[u], sems[u])

  def body(i, carry):
    e0 = nbuf * i
    for u in range(nbuf):
      wait(bufs[u], sems[u])
      reduce_into(bufs[u], e0 + u)

      @pl.when(e0 + u + nbuf < b_per_w)
      def _():
        start(e0 + u + nbuf, bufs[u], sems[u])

    return carry

  lax.fori_loop(0, b_per_w // nbuf, body, 0)

  pltpu.sync_copy(out_v, out_hbm.at[pl.ds(base, b_per_w)])


def _head_body(scale, agg_ref, wt_ref, b_ref, out_ref):
  out_ref[...] = (
      jnp.dot(agg_ref[...], wt_ref[...], preferred_element_type=jnp.float32)
      * scale + b_ref[...])


def kernel(x, table, W, b):
  B, H = x.shape
  V, D = table.shape
  OUT = W.shape[0]
  nw = NC * NS
  b_per_w = B // nw

  x_flat = x.astype(jnp.int32).reshape(-1)

  grid = (V + C - 1) // C
  packed = pl.pallas_call(
      _pack_body,
      grid=(grid,),
      in_specs=[pl.BlockSpec((D, C), lambda j: (0, j))],
      out_specs=pl.BlockSpec((C // 4, 128), lambda j: (j, 0)),
      out_shape=jax.ShapeDtypeStruct((grid * (C // 4), 128), jnp.float32),
  )(table.T)
  v_pad = grid * C
  table_lin = packed.reshape(v_pad, D)

  mesh = plsc.VectorSubcoreMesh(core_axis_name="c", subcore_axis_name="s",
                                num_cores=NC, num_subcores=NS)
  sums = pl.kernel(
      functools.partial(_pool_body, B, H, b_per_w),
      out_type=jax.ShapeDtypeStruct((B, D), jnp.float32),
      mesh=mesh,
      compiler_params=pltpu.CompilerParams(use_tc_tiling_on_sc=False),
      scratch_types=[
          pltpu.VMEM((b_per_w * H,), jnp.int32),
          pltpu.VMEM((H, D), jnp.float32),
          pltpu.VMEM((H, D), jnp.float32),
          pltpu.VMEM((H, D), jnp.float32),
          pltpu.VMEM((H, D), jnp.float32),
          pltpu.VMEM((b_per_w, D), jnp.float32),
          pltpu.SemaphoreType.DMA,
          pltpu.SemaphoreType.DMA,
          pltpu.SemaphoreType.DMA,
          pltpu.SemaphoreType.DMA,
      ],
  )(x_flat, table_lin)

  out = pl.pallas_call(
      functools.partial(_head_body, 1.0 / H),
      out_shape=jax.ShapeDtypeStruct((B, OUT), jnp.float32),
  )(sums, W.T, b.reshape(1, OUT))
  return out


# speedup vs baseline: 8.7144x; 1.0438x over previous
import functools

import jax
import jax.numpy as jnp
from jax import lax
from jax.experimental import pallas as pl
from jax.experimental.pallas import tpu as pltpu
from jax.experimental.pallas import tpu_sc as plsc

NC = 2
NS = 16
LANES = 16

S0 = 104
C = 65536


def _pack_body(tt_ref, out_ref):
  n = C // 4
  stacked = jnp.concatenate(
      [tt_ref[:, n * k:n * (k + 1)] for k in range(4)], axis=0)
  out_ref[...] = stacked.T


def _pool_body(B, H, b_per_w,
               x_hbm, table_hbm, out_hbm,
               idx_v, rows_a, rows_b, rows_c, rows_d,
               rows_e, rows_f, rows_g, rows_h, out_v,
               sem_a, sem_b, sem_c, sem_d,
               sem_e, sem_f, sem_g, sem_h):
  s1 = H - S0
  wid = lax.axis_index("s") * NC + lax.axis_index("c")
  base = wid * b_per_w

  pltpu.sync_copy(x_hbm.at[pl.ds(base * H, b_per_w * H)], idx_v)

  n = C // 4
  sh = n.bit_length() - 1

  def tbody(i, carry):
    for u in range(4):
      o = (i * 4 + u) * LANES
      v = idx_v[pl.ds(o, LANES)]
      w = v & (C - 1)
      idx_v[pl.ds(o, LANES)] = (v - w) + ((w & (n - 1)) << 2) + (w >> sh)
    return carry

  lax.fori_loop(0, (b_per_w * H) // (4 * LANES), tbody, 0)

  def start(e, rows, sem):
    off = e * H
    pltpu.async_copy(table_hbm.at[idx_v.at[pl.ds(off, S0)]],
                     rows.at[pl.ds(0, S0)], sem)
    pltpu.async_copy(table_hbm.at[idx_v.at[pl.ds(off + S0, s1)]],
                     rows.at[pl.ds(S0, s1)], sem)

  def wait(rows, sem):
    pltpu.make_async_copy(table_hbm.at[idx_v.at[pl.ds(0, S0)]],
                          rows.at[pl.ds(0, S0)], sem).wait()
    pltpu.make_async_copy(table_hbm.at[idx_v.at[pl.ds(0, s1)]],
                          rows.at[pl.ds(S0, s1)], sem).wait()

  def reduce_into(rows, e):
    zero = jnp.zeros((LANES,), jnp.float32)

    def red(i, accs):
      a0, a1, a2, a3 = accs
      for u in range(8):
        j = i * 8 + u
        if u % 2 == 0:
          a0 = a0 + rows[j, 0:16]
          a1 = a1 + rows[j, 16:32]
        else:
          a2 = a2 + rows[j, 0:16]
          a3 = a3 + rows[j, 16:32]
      return a0, a1, a2, a3

    a0, a1, a2, a3 = lax.fori_loop(0, H // 8, red, (zero, zero, zero, zero))
    out_v[e, 0:16] = a0 + a2
    out_v[e, 16:32] = a1 + a3

  bufs = (rows_a, rows_b, rows_c, rows_d, rows_e, rows_f, rows_g, rows_h)
  sems = (sem_a, sem_b, sem_c, sem_d, sem_e, sem_f, sem_g, sem_h)
  nbuf = len(bufs)
  for u in range(nbuf):
    start(u, bufs[u], sems[u])

  def body(i, carry):
    e0 = nbuf * i
    for u in range(nbuf):
      wait(bufs[u], sems[u])
      reduce_into(bufs[u], e0 + u)

      @pl.when(e0 + u + nbuf < b_per_w)
      def _():
        start(e0 + u + nbuf, bufs[u], sems[u])

    return carry

  lax.fori_loop(0, b_per_w // nbuf, body, 0)

  pltpu.sync_copy(out_v, out_hbm.at[pl.ds(base, b_per_w)])


def _head_body(scale, agg_ref, wt_ref, b_ref, out_ref):
  out_ref[...] = (
      jnp.dot(agg_ref[...], wt_ref[...], preferred_element_type=jnp.float32)
      * scale + b_ref[...])


def kernel(x, table, W, b):
  B, H = x.shape
  V, D = table.shape
  OUT = W.shape[0]
  nw = NC * NS
  b_per_w = B // nw

  x_flat = x.astype(jnp.int32).reshape(-1)

  grid = (V + C - 1) // C
  packed = pl.pallas_call(
      _pack_body,
      grid=(grid,),
      in_specs=[pl.BlockSpec((D, C), lambda j: (0, j))],
      out_specs=pl.BlockSpec((C // 4, 128), lambda j: (j, 0)),
      out_shape=jax.ShapeDtypeStruct((grid * (C // 4), 128), jnp.float32),
  )(table.T)
  v_pad = grid * C
  table_lin = packed.reshape(v_pad, D)

  mesh = plsc.VectorSubcoreMesh(core_axis_name="c", subcore_axis_name="s",
                                num_cores=NC, num_subcores=NS)
  sums = pl.kernel(
      functools.partial(_pool_body, B, H, b_per_w),
      out_type=jax.ShapeDtypeStruct((B, D), jnp.float32),
      mesh=mesh,
      compiler_params=pltpu.CompilerParams(use_tc_tiling_on_sc=False),
      scratch_types=[
          pltpu.VMEM((b_per_w * H,), jnp.int32),
          pltpu.VMEM((H, D), jnp.float32),
          pltpu.VMEM((H, D), jnp.float32),
          pltpu.VMEM((H, D), jnp.float32),
          pltpu.VMEM((H, D), jnp.float32),
          pltpu.VMEM((H, D), jnp.float32),
          pltpu.VMEM((H, D), jnp.float32),
          pltpu.VMEM((H, D), jnp.float32),
          pltpu.VMEM((H, D), jnp.float32),
          pltpu.VMEM((b_per_w, D), jnp.float32),
          pltpu.SemaphoreType.DMA,
          pltpu.SemaphoreType.DMA,
          pltpu.SemaphoreType.DMA,
          pltpu.SemaphoreType.DMA,
          pltpu.SemaphoreType.DMA,
          pltpu.SemaphoreType.DMA,
          pltpu.SemaphoreType.DMA,
          pltpu.SemaphoreType.DMA,
      ],
  )(x_flat, table_lin)

  out = pl.pallas_call(
      functools.partial(_head_body, 1.0 / H),
      out_shape=jax.ShapeDtypeStruct((B, OUT), jnp.float32),
  )(sums, W.T, b.reshape(1, OUT))
  return out


# speedup vs baseline: 8.8181x; 1.0119x over previous
import functools

import jax
import jax.numpy as jnp
from jax import lax
from jax.experimental import pallas as pl
from jax.experimental.pallas import tpu as pltpu
from jax.experimental.pallas import tpu_sc as plsc

NC = 2
NS = 16
LANES = 16

S0 = 104
C = 65536


def _pack_body(tt_ref, out_ref):
  n = C // 4
  stacked = jnp.concatenate(
      [tt_ref[:, n * k:n * (k + 1)] for k in range(4)], axis=0)
  out_ref[...] = stacked.T


def _pool_body(B, H, b_per_w,
               x_hbm, table_hbm, out_hbm,
               idx_v, rows_a, rows_b, rows_c, rows_d,
               rows_e, rows_f, rows_g, rows_h, out_v,
               sem_a, sem_b, sem_c, sem_d,
               sem_e, sem_f, sem_g, sem_h):
  s1 = H - S0
  wid = lax.axis_index("s") * NC + lax.axis_index("c")
  base = wid * b_per_w

  pltpu.sync_copy(x_hbm.at[pl.ds(base * H, b_per_w * H)], idx_v)

  n = C // 4
  sh = n.bit_length() - 1

  def tbody(i, carry):
    for u in range(4):
      o = (i * 4 + u) * LANES
      v = idx_v[pl.ds(o, LANES)]
      w = v & (C - 1)
      idx_v[pl.ds(o, LANES)] = (v - w) + ((w & (n - 1)) << 2) + (w >> sh)
    return carry

  head_iters = (8 * H + 4 * LANES - 1) // (4 * LANES)
  lax.fori_loop(0, head_iters, tbody, 0)

  def start(e, rows, sem):
    off = e * H
    pltpu.async_copy(table_hbm.at[idx_v.at[pl.ds(off, S0)]],
                     rows.at[pl.ds(0, S0)], sem)
    pltpu.async_copy(table_hbm.at[idx_v.at[pl.ds(off + S0, s1)]],
                     rows.at[pl.ds(S0, s1)], sem)

  def wait(rows, sem):
    pltpu.make_async_copy(table_hbm.at[idx_v.at[pl.ds(0, S0)]],
                          rows.at[pl.ds(0, S0)], sem).wait()
    pltpu.make_async_copy(table_hbm.at[idx_v.at[pl.ds(0, s1)]],
                          rows.at[pl.ds(S0, s1)], sem).wait()

  def reduce_into(rows, e):
    zero = jnp.zeros((LANES,), jnp.float32)

    def red(i, accs):
      a0, a1, a2, a3 = accs
      for u in range(8):
        j = i * 8 + u
        if u % 2 == 0:
          a0 = a0 + rows[j, 0:16]
          a1 = a1 + rows[j, 16:32]
        else:
          a2 = a2 + rows[j, 0:16]
          a3 = a3 + rows[j, 16:32]
      return a0, a1, a2, a3

    a0, a1, a2, a3 = lax.fori_loop(0, H // 8, red, (zero, zero, zero, zero))
    out_v[e, 0:16] = a0 + a2
    out_v[e, 16:32] = a1 + a3

  bufs = (rows_a, rows_b, rows_c, rows_d, rows_e, rows_f, rows_g, rows_h)
  sems = (sem_a, sem_b, sem_c, sem_d, sem_e, sem_f, sem_g, sem_h)
  nbuf = len(bufs)
  for u in range(nbuf):
    start(u, bufs[u], sems[u])

  lax.fori_loop(head_iters, (b_per_w * H) // (4 * LANES), tbody, 0)

  def body(i, carry):
    e0 = nbuf * i
    for u in range(nbuf):
      wait(bufs[u], sems[u])
      reduce_into(bufs[u], e0 + u)

      @pl.when(e0 + u + nbuf < b_per_w)
      def _():
        start(e0 + u + nbuf, bufs[u], sems[u])

    return carry

  lax.fori_loop(0, b_per_w // nbuf, body, 0)

  pltpu.sync_copy(out_v, out_hbm.at[pl.ds(base, b_per_w)])


def _head_body(scale, agg_ref, wt_ref, b_ref, out_ref):
  out_ref[...] = (
      jnp.dot(agg_ref[...], wt_ref[...], preferred_element_type=jnp.float32)
      * scale + b_ref[...])


def kernel(x, table, W, b):
  B, H = x.shape
  V, D = table.shape
  OUT = W.shape[0]
  nw = NC * NS
  b_per_w = B // nw

  x_flat = x.astype(jnp.int32).reshape(-1)

  grid = (V + C - 1) // C
  packed = pl.pallas_call(
      _pack_body,
      grid=(grid,),
      in_specs=[pl.BlockSpec((D, C), lambda j: (0, j))],
      out_specs=pl.BlockSpec((C // 4, 128), lambda j: (j, 0)),
      out_shape=jax.ShapeDtypeStruct((grid * (C // 4), 128), jnp.float32),
  )(table.T)
  v_pad = grid * C
  table_lin = packed.reshape(v_pad, D)

  mesh = plsc.VectorSubcoreMesh(core_axis_name="c", subcore_axis_name="s",
                                num_cores=NC, num_subcores=NS)
  sums = pl.kernel(
      functools.partial(_pool_body, B, H, b_per_w),
      out_type=jax.ShapeDtypeStruct((B, D), jnp.float32),
      mesh=mesh,
      compiler_params=pltpu.CompilerParams(use_tc_tiling_on_sc=False),
      scratch_types=[
          pltpu.VMEM((b_per_w * H,), jnp.int32),
          pltpu.VMEM((H, D), jnp.float32),
          pltpu.VMEM((H, D), jnp.float32),
          pltpu.VMEM((H, D), jnp.float32),
          pltpu.VMEM((H, D), jnp.float32),
          pltpu.VMEM((H, D), jnp.float32),
          pltpu.VMEM((H, D), jnp.float32),
          pltpu.VMEM((H, D), jnp.float32),
          pltpu.VMEM((H, D), jnp.float32),
          pltpu.VMEM((b_per_w, D), jnp.float32),
          pltpu.SemaphoreType.DMA,
          pltpu.SemaphoreType.DMA,
          pltpu.SemaphoreType.DMA,
          pltpu.SemaphoreType.DMA,
          pltpu.SemaphoreType.DMA,
          pltpu.SemaphoreType.DMA,
          pltpu.SemaphoreType.DMA,
          pltpu.SemaphoreType.DMA,
      ],
  )(x_flat, table_lin)

  out = pl.pallas_call(
      functools.partial(_head_body, 1.0 / H),
      out_shape=jax.ShapeDtypeStruct((B, OUT), jnp.float32),
  )(sums, W.T, b.reshape(1, OUT))
  return out
